# baseline (speedup 1.0000x reference)
import functools

import jax
import jax.numpy as jnp
from jax.experimental import pallas as pl
from jax.experimental.pallas import tpu as pltpu


def _mlp_kernel(x_ref, w1_ref, b1_ref, w2_ref, b2_ref, o_ref, h_ref, *, ne):
    e = pl.program_id(0)

    @pl.when(e > 0)
    def _layer2():
        hb = h_ref[jax.lax.rem(e + 1, 2)]
        o = jnp.dot(hb, w2_ref[0], preferred_element_type=jnp.float32)
        o_ref[0] = o + b2_ref[0]

    @pl.when(e < ne)
    def _layer1():
        h = jnp.dot(x_ref[0], w1_ref[0], preferred_element_type=jnp.float32)
        h_ref[jax.lax.rem(e, 2)] = jnp.maximum(h + b1_ref[0], 0.0).astype(
            jnp.bfloat16)


def kernel(x, W1, b1, W2, b2):
    B, S, D = x.shape
    E, _, F = W1.shape
    T = B * S
    per = T // E
    xr = x.reshape(E, per, D).astype(jnp.bfloat16)
    w1b = W1.astype(jnp.bfloat16)
    w2b = W2.astype(jnp.bfloat16)
    last = E - 1
    out = pl.pallas_call(
        functools.partial(_mlp_kernel, ne=E),
        grid=(E + 1,),
        in_specs=[
            pl.BlockSpec((1, per, D), lambda e: (jnp.minimum(e, last), 0, 0)),
            pl.BlockSpec((1, D, F), lambda e: (jnp.minimum(e, last), 0, 0)),
            pl.BlockSpec((1, 1, F), lambda e: (jnp.minimum(e, last), 0, 0)),
            pl.BlockSpec((1, F, D), lambda e: (jnp.maximum(e - 1, 0), 0, 0)),
            pl.BlockSpec((1, 1, D), lambda e: (jnp.maximum(e - 1, 0), 0, 0)),
        ],
        out_specs=pl.BlockSpec((1, per, D),
                               lambda e: (jnp.maximum(e - 1, 0), 0, 0)),
        out_shape=jax.ShapeDtypeStruct((E, per, D), x.dtype),
        scratch_shapes=[pltpu.VMEM((2, per, F), jnp.bfloat16)],
        compiler_params=pltpu.CompilerParams(
            dimension_semantics=("arbitrary",),
        ),
    )(xr, w1b, b1.reshape(E, 1, F), w2b, b2.reshape(E, 1, D))
    return out.reshape(B, S, D)


# speedup vs baseline: 1.5027x; 1.5027x over previous
import functools

import jax
import jax.numpy as jnp
from jax.experimental import pallas as pl
from jax.experimental.pallas import tpu as pltpu


def _mlp_kernel(x_ref, w1_ref, b1_ref, w2_ref, b2_ref, o_ref, h_ref, *,
                ne, bt):
    e = pl.program_id(0)
    t = pl.program_id(1)

    @pl.when(e > 0)
    def _layer2():
        hb = h_ref[jax.lax.rem(e + 1, 2), pl.ds(t * bt, bt)]
        w2b = w2_ref[0].astype(jnp.bfloat16)
        o = jnp.dot(hb, w2b, preferred_element_type=jnp.float32)
        o_ref[0] = o + b2_ref[0]

    @pl.when(e < ne)
    def _layer1():
        h = jnp.dot(x_ref[0], w1_ref[0], preferred_element_type=jnp.float32)
        h_ref[jax.lax.rem(e, 2), pl.ds(t * bt, bt)] = jnp.maximum(
            h + b1_ref[0], 0.0).astype(jnp.bfloat16)


def kernel(x, W1, b1, W2, b2):
    B, S, D = x.shape
    E, _, F = W1.shape
    T = B * S
    per = T // E
    nt = 2
    bt = per // nt
    xr = x.reshape(E * nt, bt, D)
    last = E - 1
    out = pl.pallas_call(
        functools.partial(_mlp_kernel, ne=E, bt=bt),
        grid=(E + 1, nt),
        in_specs=[
            pl.BlockSpec((1, bt, D),
                         lambda e, t: (jnp.minimum(e, last) * nt + t, 0, 0)),
            pl.BlockSpec((1, D, F), lambda e, t: (jnp.minimum(e, last), 0, 0)),
            pl.BlockSpec((1, 1, F), lambda e, t: (jnp.minimum(e, last), 0, 0)),
            pl.BlockSpec((1, F, D), lambda e, t: (jnp.maximum(e - 1, 0), 0, 0)),
            pl.BlockSpec((1, 1, D), lambda e, t: (jnp.maximum(e - 1, 0), 0, 0)),
        ],
        out_specs=pl.BlockSpec(
            (1, bt, D), lambda e, t: (jnp.maximum(e - 1, 0) * nt + t, 0, 0)),
        out_shape=jax.ShapeDtypeStruct((E * nt, bt, D), x.dtype),
        scratch_shapes=[pltpu.VMEM((2, per, F), jnp.bfloat16)],
        compiler_params=pltpu.CompilerParams(
            dimension_semantics=("arbitrary", "arbitrary"),
        ),
    )(xr, W1, b1.reshape(E, 1, F), W2, b2.reshape(E, 1, D))
    return out.reshape(B, S, D)


# speedup vs baseline: 1.8362x; 1.2220x over previous
import functools

import jax
import jax.numpy as jnp
from jax.experimental import pallas as pl
from jax.experimental.pallas import tpu as pltpu


def _mlp_kernel(x_ref, w1_ref, b1_ref, w2_ref, b2_ref, o_ref, h_ref, *, ns):
    s = pl.program_id(0)

    @pl.when(s > 0)
    def _layer2():
        hb = h_ref[jax.lax.rem(s + 1, 2)]
        w2b = w2_ref[0].astype(jnp.bfloat16)
        o = jnp.dot(hb, w2b, preferred_element_type=jnp.float32)
        o_ref[0] = o + b2_ref[0]

    @pl.when(s < ns)
    def _layer1():
        h = jnp.dot(x_ref[0], w1_ref[0], preferred_element_type=jnp.float32)
        h_ref[jax.lax.rem(s, 2)] = jnp.maximum(h + b1_ref[0], 0.0).astype(
            jnp.bfloat16)


def kernel(x, W1, b1, W2, b2):
    B, S, D = x.shape
    E, _, F = W1.shape
    T = B * S
    per = T // E
    nt = 2
    bt = per // nt
    en = E * nt
    last = en - 1
    xr = x.reshape(en, bt, D)
    out = pl.pallas_call(
        functools.partial(_mlp_kernel, ns=en),
        grid=(en + 1,),
        in_specs=[
            pl.BlockSpec((1, bt, D), lambda s: (jnp.minimum(s, last), 0, 0)),
            pl.BlockSpec((1, D, F),
                         lambda s: (jnp.minimum(s, last) // nt, 0, 0)),
            pl.BlockSpec((1, 1, F),
                         lambda s: (jnp.minimum(s, last) // nt, 0, 0)),
            pl.BlockSpec((1, F, D),
                         lambda s: (jnp.maximum(s - 1, 0) // nt, 0, 0)),
            pl.BlockSpec((1, 1, D),
                         lambda s: (jnp.maximum(s - 1, 0) // nt, 0, 0)),
        ],
        out_specs=pl.BlockSpec((1, bt, D),
                               lambda s: (jnp.maximum(s - 1, 0), 0, 0)),
        out_shape=jax.ShapeDtypeStruct((en, bt, D), x.dtype),
        scratch_shapes=[pltpu.VMEM((2, bt, F), jnp.bfloat16)],
        compiler_params=pltpu.CompilerParams(
            dimension_semantics=("arbitrary",),
        ),
    )(xr, W1, b1.reshape(E, 1, F), W2, b2.reshape(E, 1, D))
    return out.reshape(B, S, D)
